# Initial kernel scaffold; baseline (speedup 1.0000x reference)
import jax
import jax.numpy as jnp
from jax.experimental import pallas as pl


def kernel(input, weight):
    raise NotImplementedError("write your pallas kernel here")


# speedup vs baseline: 1.5765x; 1.5765x over previous
import functools

import jax
import jax.numpy as jnp
from jax import lax
from jax.experimental import pallas as pl
from jax.experimental.pallas import tpu as pltpu
from jax.experimental.pallas import tpu_sc as plsc

_NC = 2
_NS = 16
_NW = _NC * _NS


@functools.lru_cache(maxsize=None)
def _build_gather(total, d, chunk):
    b_per_w = total // _NW
    n_chunks = b_per_w // chunk
    assert b_per_w % chunk == 0 and b_per_w % 8 == 0

    mesh = plsc.VectorSubcoreMesh(core_axis_name="c", subcore_axis_name="s")

    @functools.partial(
        pl.kernel,
        mesh=mesh,
        out_type=jax.ShapeDtypeStruct((total, d), jnp.float32),
        scratch_types=[
            pltpu.VMEM((b_per_w,), jnp.int32),
            pltpu.VMEM((chunk, d), jnp.float32),
            pltpu.SemaphoreType.DMA,
        ],
        compiler_params=pltpu.CompilerParams(use_tc_tiling_on_sc=False),
    )
    def gather_kernel(table_hbm, idx_hbm, out_hbm, idx_v, rows_v, sem):
        wid = lax.axis_index("s") * _NC + lax.axis_index("c")
        base = wid * b_per_w
        pltpu.sync_copy(idx_hbm.at[pl.ds(base, b_per_w)], idx_v)
        for c in range(n_chunks):
            pltpu.async_copy(
                table_hbm.at[idx_v.at[pl.ds(c * chunk, chunk)]], rows_v, sem
            ).wait()
            pltpu.sync_copy(rows_v, out_hbm.at[pl.ds(base + c * chunk, chunk)])

    return gather_kernel


def kernel(input, weight):
    b, f = input.shape
    _, d = weight.shape
    total = b * f
    idx = input.reshape(total).astype(jnp.int32)
    out = _build_gather(total, d, chunk=3328)(weight, idx)
    return out.reshape(b, f, d)


# speedup vs baseline: 1.5842x; 1.0049x over previous
import functools

import jax
import jax.numpy as jnp
from jax import lax
from jax.experimental import pallas as pl
from jax.experimental.pallas import tpu as pltpu
from jax.experimental.pallas import tpu_sc as plsc

_NC = 2
_NS = 16
_NW = _NC * _NS


@functools.lru_cache(maxsize=None)
def _build_gather(total, d, chunk, nbuf):
    b_per_w = total // _NW
    n_chunks = b_per_w // chunk
    assert b_per_w % chunk == 0 and b_per_w % 8 == 0 and n_chunks >= nbuf

    mesh = plsc.VectorSubcoreMesh(core_axis_name="c", subcore_axis_name="s")

    @functools.partial(
        pl.kernel,
        mesh=mesh,
        out_type=jax.ShapeDtypeStruct((total, d), jnp.float32),
        scratch_types=[
            pltpu.VMEM((b_per_w,), jnp.int32),
            [pltpu.VMEM((chunk, d), jnp.float32) for _ in range(nbuf)],
            pltpu.SemaphoreType.DMA,
            pltpu.SemaphoreType.DMA,
        ],
        compiler_params=pltpu.CompilerParams(use_tc_tiling_on_sc=False),
    )
    def gather_kernel(table_hbm, idx_hbm, out_hbm, idx_v, rows_bufs, gsem, ssem):
        wid = lax.axis_index("s") * _NC + lax.axis_index("c")
        base = wid * b_per_w
        pltpu.sync_copy(idx_hbm.at[pl.ds(base, b_per_w)], idx_v)

        def start_gather(c):
            return pltpu.async_copy(
                table_hbm.at[idx_v.at[pl.ds(c * chunk, chunk)]],
                rows_bufs[c % nbuf],
                gsem,
            )

        gathers = [None] * n_chunks
        stores = [None] * n_chunks
        for c in range(min(nbuf, n_chunks)):
            gathers[c] = start_gather(c)
        for c in range(n_chunks):
            gathers[c].wait()
            stores[c] = pltpu.async_copy(
                rows_bufs[c % nbuf], out_hbm.at[pl.ds(base + c * chunk, chunk)], ssem
            )
            nxt = c + nbuf
            if nxt < n_chunks:
                stores[c].wait()
                gathers[nxt] = start_gather(nxt)
        for c in range(max(0, n_chunks - nbuf), n_chunks):
            stores[c].wait()

    return gather_kernel


def kernel(input, weight):
    b, f = input.shape
    _, d = weight.shape
    total = b * f
    idx = input.reshape(total).astype(jnp.int32)
    out = _build_gather(total, d, chunk=1664, nbuf=2)(weight, idx)
    return out.reshape(b, f, d)
